# baseline (speedup 1.0000x reference)
import functools

import jax
import jax.numpy as jnp
from jax import lax
from jax.experimental import pallas as pl
from jax.experimental.pallas import tpu as pltpu
from jax.experimental.pallas import tpu_sc as plsc

B = 16384
D = 32
H = 128
NC = 2
NS = 16
NW = NC * NS
BPW = B // NW

NU = 1000000
CW = 896
NFULL = NU // CW
PART_R0 = NFULL * CW
PART_W = NU - PART_R0
W_HI = 28
CPW_HI = 35
CPW_LO = 34
PAIRS = (CPW_HI + 1) // 2
CAP = 128
TRASH = B
LCAP = B + 64
SENTINEL = 0x7FFFFFF0


def _iota16():
    return lax.iota(jnp.int32, 16)


def _full16(v):
    return jnp.full((16,), v, jnp.int32)


def _sc_user_body(u_idx, ut, out, uidx_v, jl, rl, chunk0, chunk1, partbuf,
                  staging, jslots, sem_c0, sem_c1, sem_s):
    wid = lax.axis_index("s") * NC + lax.axis_index("c")
    chunk_lo = jnp.where(wid < W_HI, wid * CPW_HI,
                         W_HI * CPW_HI + (wid - W_HI) * CPW_LO)
    n_full = jnp.where(wid < W_HI, CPW_HI, CPW_LO)
    r_lo = chunk_lo * CW
    r_hi = r_lo + n_full * CW + jnp.where(wid == NW - 1, PART_W, 0)
    bufs = (chunk0, chunk1)

    sems = (sem_c0, sem_c1)
    pltpu.async_copy(ut.at[:, pl.ds(chunk_lo * CW, CW)], chunk0, sem_c0)
    pltpu.async_copy(ut.at[:, pl.ds((chunk_lo + 1) * CW, CW)], chunk1, sem_c1)

    pltpu.sync_copy(u_idx, uidx_v)

    def sel_body(c4, cnt):
        for k in range(4):
            c = c4 * 4 + k
            v = uidx_v[pl.ds(c * 16, 16)]
            j = _iota16() + c * 16
            m = (v >= r_lo) & (v < r_hi)
            plsc.store_compressed(jl.at[pl.ds(cnt, 16)], j, mask=m)
            plsc.store_compressed(rl.at[pl.ds(cnt, 16)], v, mask=m)
            cnt = cnt + plsc.all_reduce_population_count(m)[0]
        return cnt

    cnt_list = lax.fori_loop(0, B // 64, sel_body, jnp.int32(0))
    for q in range(4):
        rl[pl.ds(cnt_list + q * 16, 16)] = _full16(SENTINEL)

    for q in range(CAP // 16):
        jslots[pl.ds(q * 16, 16)] = _full16(TRASH)

    def flush():
        pltpu.async_copy(staging, out.at[jslots], sem_s).wait()
        for q in range(CAP // 16):
            jslots[pl.ds(q * 16, 16)] = _full16(TRASH)

    ngroups4 = (cnt_list + 63) // 64

    def extract(buf, width, r0, cs):
        def gbody(g4, cs):
            for k in range(4):
                g = g4 * 4 + k
                rv = rl[pl.ds(g * 16, 16)]
                m = (rv >= r0) & (rv < r0 + width)
                npc = plsc.all_reduce_population_count(m)[0]

                @pl.when(npc > 0)
                def _():
                    jv = jl[pl.ds(g * 16, 16)]
                    pos = plsc.cumsum(jnp.where(m, 1, 0).astype(jnp.int32))
                    slots = cs + pos - 1
                    col = jnp.where(m, rv - r0, 0)
                    for d in range(D):
                        vals = plsc.load_gather(buf, [_full16(d), col],
                                                mask=m)
                        plsc.store_scatter(staging, [slots, _full16(d)],
                                           vals, mask=m)
                    plsc.store_scatter(jslots, [slots], jv, mask=m)

                cs = cs + npc

            def do_flush(c):
                flush()
                return jnp.int32(0)

            return lax.cond(cs >= CAP - 64, do_flush, lambda c: c, cs)

        return lax.fori_loop(0, ngroups4, gbody, cs)

    def pair_body(p, cs):
        for b in range(2):
            c = 2 * p + b
            buf = bufs[b]
            sem_b = sems[b]

            def process(cs):
                pltpu.make_async_copy(
                    ut.at[:, pl.ds(0, CW)], buf, sem_b).wait()
                cs = extract(buf, CW, (chunk_lo + c) * CW, cs)

                @pl.when(c + 2 < n_full)
                def _():
                    pltpu.async_copy(
                        ut.at[:, pl.ds((chunk_lo + c + 2) * CW, CW)],
                        buf, sem_b)

                return cs

            cs = lax.cond(c < n_full, process, lambda c_: c_, cs)
        return cs

    cs = lax.fori_loop(0, PAIRS, pair_body, jnp.int32(0))

    def do_partial(cs):
        pltpu.sync_copy(ut.at[:, pl.ds(PART_R0, PART_W)], partbuf)
        return extract(partbuf, PART_W, jnp.int32(PART_R0), cs)

    cs = lax.cond(wid == NW - 1, do_partial, lambda c: c, cs)
    flush()


def _sc_user(u_idx, ut):
    mesh = plsc.VectorSubcoreMesh(core_axis_name="c", subcore_axis_name="s")
    scratch = [
        pltpu.VMEM((B,), jnp.int32),
        pltpu.VMEM((LCAP,), jnp.int32),
        pltpu.VMEM((LCAP,), jnp.int32),
        pltpu.VMEM((D, CW), jnp.float32),
        pltpu.VMEM((D, CW), jnp.float32),
        pltpu.VMEM((D, PART_W), jnp.float32),
        pltpu.VMEM((CAP, H), jnp.float32),
        pltpu.VMEM((CAP,), jnp.int32),
        pltpu.SemaphoreType.DMA,
        pltpu.SemaphoreType.DMA,
        pltpu.SemaphoreType.DMA,
    ]
    k = pl.kernel(
        _sc_user_body,
        out_type=jax.ShapeDtypeStruct((B + 2048, H), jnp.float32),
        mesh=mesh,
        scratch_types=scratch,
        compiler_params=pltpu.CompilerParams(needs_layout_passes=False),
    )
    return k(u_idx, ut)


def _sc_small_body(i_idx, l_idx, c_idx, it, lt, ct, out, idxb, rows, sem):
    wid = lax.axis_index("s") * NC + lax.axis_index("c")
    base = wid * BPW
    idxs = (i_idx, l_idx, c_idx)
    tables = (it, lt, ct)
    for t in range(3):
        pltpu.sync_copy(idxs[t].at[pl.ds(base, BPW)], idxb)
        pltpu.async_copy(tables[t].at[idxb], rows, sem).wait()
        pltpu.sync_copy(rows, out.at[t, pl.ds(base, BPW)])


def _sc_small(i_idx, l_idx, c_idx, it, lt, ct):
    mesh = plsc.VectorSubcoreMesh(core_axis_name="c", subcore_axis_name="s")
    scratch = [
        pltpu.VMEM((BPW,), jnp.int32),
        pltpu.VMEM((BPW, D), jnp.float32),
        pltpu.SemaphoreType.DMA,
    ]
    k = pl.kernel(
        _sc_small_body,
        out_type=jax.ShapeDtypeStruct((3, B, D), jnp.float32),
        mesh=mesh,
        scratch_types=scratch,
        compiler_params=pltpu.CompilerParams(use_tc_tiling_on_sc=False),
    )
    return k(i_idx, l_idx, c_idx, it, lt, ct)


BM = 2048


def _mlp_body(gu_ref, g3_ref, w1_ref, b1_ref, w2_ref, b2_ref, out_ref):
    h = jnp.dot(gu_ref[:, 0:D], w1_ref[0:D, :],
                preferred_element_type=jnp.float32)
    for t in range(3):
        h = h + jnp.dot(g3_ref[t], w1_ref[(t + 1) * D:(t + 2) * D, :],
                        preferred_element_type=jnp.float32)
    h = jnp.maximum(h + b1_ref[0, :][None, :], 0.0)
    out_ref[...] = (
        jnp.sum(h * w2_ref[0, :][None, :], axis=1, keepdims=True)
        + b2_ref[0, 0]
    )


def _mlp(gu, g3, W1, b1, W2, b2):
    w2_row = W2.reshape(1, H)
    b1_row = b1.reshape(1, H)
    b2_s = b2.reshape(1, 1)
    out = pl.pallas_call(
        _mlp_body,
        grid=(B // BM,),
        in_specs=[
            pl.BlockSpec((BM, H), lambda i: (i, 0)),
            pl.BlockSpec((3, BM, D), lambda i: (0, i, 0)),
            pl.BlockSpec((H, H), lambda i: (0, 0)),
            pl.BlockSpec((1, H), lambda i: (0, 0)),
            pl.BlockSpec((1, H), lambda i: (0, 0)),
            pl.BlockSpec((1, 1), lambda i: (0, 0)),
        ],
        out_specs=pl.BlockSpec((BM, 1), lambda i: (i, 0)),
        out_shape=jax.ShapeDtypeStruct((B, 1), jnp.float32),
    )(gu, g3, W1, b1_row, w2_row, b2_s)
    return out[:, 0]


def kernel(user, item, language, category,
           user_emb, item_emb, language_emb, category_emb,
           W1, b1, W2, b2):
    user = user.astype(jnp.int32)
    item = item.astype(jnp.int32)
    language = language.astype(jnp.int32)
    category = category.astype(jnp.int32)
    gu = _sc_user(user, user_emb.T)
    g3 = _sc_small(item, language, category,
                   item_emb, language_emb, category_emb)
    return _mlp(gu, g3, W1, b1, W2, b2)


# speedup vs baseline: 2.3102x; 2.3102x over previous
import functools

import jax
import jax.numpy as jnp
from jax import lax
from jax.experimental import pallas as pl
from jax.experimental.pallas import tpu as pltpu
from jax.experimental.pallas import tpu_sc as plsc

B = 16384
D = 32
H = 128
NC = 2
NS = 16
NW = NC * NS
BPW = B // NW

NU = 1000000
CW = 768
NFULL = NU // CW
PART_R0 = NFULL * CW
PART_W = NU - PART_R0
W_HI = 22
CPW_HI = 41
CPW_LO = 40
PAIRS = (CPW_HI + 1) // 2
CAP = 96
TRASH = B
LCAP = B + 16


def _iota16():
    return lax.iota(jnp.int32, 16)


def _full16(v):
    return jnp.full((16,), v, jnp.int32)


def _sc_user_body(u_idx, ut, out, uidx_v, jl, rl, chunk0, chunk1, partbuf,
                  staging, jslots, sem_c0, sem_c1, sem_s):
    wid = lax.axis_index("s") * NC + lax.axis_index("c")
    chunk_lo = jnp.where(wid < W_HI, wid * CPW_HI,
                         W_HI * CPW_HI + (wid - W_HI) * CPW_LO)
    n_full = jnp.where(wid < W_HI, CPW_HI, CPW_LO)
    r_lo = chunk_lo * CW
    r_hi = r_lo + n_full * CW + jnp.where(wid == NW - 1, PART_W, 0)
    bufs = (chunk0, chunk1)

    sems = (sem_c0, sem_c1)
    pltpu.async_copy(ut.at[:, pl.ds(chunk_lo * CW, CW)], chunk0, sem_c0)
    pltpu.async_copy(ut.at[:, pl.ds((chunk_lo + 1) * CW, CW)], chunk1, sem_c1)

    pltpu.sync_copy(u_idx, uidx_v)

    def sel_body(c, cnt):
        v = uidx_v[pl.ds(c * 16, 16)]
        j = _iota16() + c * 16
        m = (v >= r_lo) & (v < r_hi)
        plsc.store_compressed(jl.at[pl.ds(cnt, 16)], j, mask=m)
        plsc.store_compressed(rl.at[pl.ds(cnt, 16)], v, mask=m)
        return cnt + plsc.all_reduce_population_count(m)[0]

    cnt_list = lax.fori_loop(0, B // 16, sel_body, jnp.int32(0))

    for q in range(CAP // 16):
        jslots[pl.ds(q * 16, 16)] = _full16(TRASH)

    def flush():
        pltpu.async_copy(staging, out.at[jslots], sem_s).wait()
        for q in range(CAP // 16):
            jslots[pl.ds(q * 16, 16)] = _full16(TRASH)

    ngroups = (cnt_list + 15) // 16

    def extract(buf, width, r0, cs):
        def gbody(g, cs):
            rv = rl[pl.ds(g * 16, 16)]
            lane = _iota16() + g * 16
            m = (rv >= r0) & (rv < r0 + width) & (lane < cnt_list)
            npc = plsc.all_reduce_population_count(m)[0]

            @pl.when(npc > 0)
            def _():
                jv = jl[pl.ds(g * 16, 16)]
                pos = plsc.cumsum(jnp.where(m, 1, 0).astype(jnp.int32))
                slots = cs + pos - 1
                col = jnp.where(m, rv - r0, 0)
                for d in range(D):
                    vals = plsc.load_gather(buf, [_full16(d), col], mask=m)
                    plsc.store_scatter(staging, [slots, _full16(d)], vals,
                                       mask=m)
                plsc.store_scatter(jslots, [slots], jv, mask=m)

            cs = cs + npc

            def do_flush(c):
                flush()
                return jnp.int32(0)

            return lax.cond(cs >= CAP - 16, do_flush, lambda c: c, cs)

        return lax.fori_loop(0, ngroups, gbody, cs)

    def pair_body(p, cs):
        for b in range(2):
            c = 2 * p + b
            buf = bufs[b]
            sem_b = sems[b]

            def process(cs):
                pltpu.make_async_copy(
                    ut.at[:, pl.ds(0, CW)], buf, sem_b).wait()
                cs = extract(buf, CW, (chunk_lo + c) * CW, cs)

                @pl.when(c + 2 < n_full)
                def _():
                    pltpu.async_copy(
                        ut.at[:, pl.ds((chunk_lo + c + 2) * CW, CW)],
                        buf, sem_b)

                return cs

            cs = lax.cond(c < n_full, process, lambda c_: c_, cs)
        return cs

    cs = lax.fori_loop(0, PAIRS, pair_body, jnp.int32(0))

    def do_partial(cs):
        pltpu.sync_copy(ut.at[:, pl.ds(PART_R0, PART_W)], partbuf)
        return extract(partbuf, PART_W, jnp.int32(PART_R0), cs)

    cs = lax.cond(wid == NW - 1, do_partial, lambda c: c, cs)
    flush()


def _sc_user(u_idx, ut):
    mesh = plsc.VectorSubcoreMesh(core_axis_name="c", subcore_axis_name="s")
    scratch = [
        pltpu.VMEM((B,), jnp.int32),
        pltpu.VMEM((LCAP,), jnp.int32),
        pltpu.VMEM((LCAP,), jnp.int32),
        pltpu.VMEM((D, CW), jnp.float32),
        pltpu.VMEM((D, CW), jnp.float32),
        pltpu.VMEM((D, PART_W), jnp.float32),
        pltpu.VMEM((CAP, H), jnp.float32),
        pltpu.VMEM((CAP,), jnp.int32),
        pltpu.SemaphoreType.DMA,
        pltpu.SemaphoreType.DMA,
        pltpu.SemaphoreType.DMA,
    ]
    k = pl.kernel(
        _sc_user_body,
        out_type=jax.ShapeDtypeStruct((B + 2048, H), jnp.float32),
        mesh=mesh,
        scratch_types=scratch,
        compiler_params=pltpu.CompilerParams(needs_layout_passes=False),
    )
    return k(u_idx, ut)


def _sc_small_body(i_idx, l_idx, c_idx, it, lt, ct, out, idxb, rows, sem):
    wid = lax.axis_index("s") * NC + lax.axis_index("c")
    base = wid * BPW
    idxs = (i_idx, l_idx, c_idx)
    tables = (it, lt, ct)
    for t in range(3):
        pltpu.sync_copy(idxs[t].at[pl.ds(base, BPW)], idxb)
        pltpu.async_copy(tables[t].at[idxb], rows, sem).wait()
        pltpu.sync_copy(rows, out.at[t, pl.ds(base, BPW)])


def _sc_small(i_idx, l_idx, c_idx, it, lt, ct):
    mesh = plsc.VectorSubcoreMesh(core_axis_name="c", subcore_axis_name="s")
    scratch = [
        pltpu.VMEM((BPW,), jnp.int32),
        pltpu.VMEM((BPW, D), jnp.float32),
        pltpu.SemaphoreType.DMA,
    ]
    k = pl.kernel(
        _sc_small_body,
        out_type=jax.ShapeDtypeStruct((3, B, D), jnp.float32),
        mesh=mesh,
        scratch_types=scratch,
        compiler_params=pltpu.CompilerParams(use_tc_tiling_on_sc=False),
    )
    return k(i_idx, l_idx, c_idx, it, lt, ct)


BM = 2048


def _mlp_body(gu_ref, g3_ref, w1_ref, b1_ref, w2_ref, b2_ref, out_ref):
    h = jnp.dot(gu_ref[:, 0:D], w1_ref[0:D, :],
                preferred_element_type=jnp.float32)
    for t in range(3):
        h = h + jnp.dot(g3_ref[t], w1_ref[(t + 1) * D:(t + 2) * D, :],
                        preferred_element_type=jnp.float32)
    h = jnp.maximum(h + b1_ref[0, :][None, :], 0.0)
    out_ref[...] = (
        jnp.sum(h * w2_ref[0, :][None, :], axis=1, keepdims=True)
        + b2_ref[0, 0]
    )


def _mlp(gu, g3, W1, b1, W2, b2):
    w2_row = W2.reshape(1, H)
    b1_row = b1.reshape(1, H)
    b2_s = b2.reshape(1, 1)
    out = pl.pallas_call(
        _mlp_body,
        grid=(B // BM,),
        in_specs=[
            pl.BlockSpec((BM, H), lambda i: (i, 0)),
            pl.BlockSpec((3, BM, D), lambda i: (0, i, 0)),
            pl.BlockSpec((H, H), lambda i: (0, 0)),
            pl.BlockSpec((1, H), lambda i: (0, 0)),
            pl.BlockSpec((1, H), lambda i: (0, 0)),
            pl.BlockSpec((1, 1), lambda i: (0, 0)),
        ],
        out_specs=pl.BlockSpec((BM, 1), lambda i: (i, 0)),
        out_shape=jax.ShapeDtypeStruct((B, 1), jnp.float32),
    )(gu, g3, W1, b1_row, w2_row, b2_s)
    return out[:, 0]


def kernel(user, item, language, category,
           user_emb, item_emb, language_emb, category_emb,
           W1, b1, W2, b2):
    user = user.astype(jnp.int32)
    item = item.astype(jnp.int32)
    language = language.astype(jnp.int32)
    category = category.astype(jnp.int32)
    gu = _sc_user(user, user_emb.T)
    g3 = _sc_small(item, language, category,
                   item_emb, language_emb, category_emb)
    return _mlp(gu, g3, W1, b1, W2, b2)


# speedup vs baseline: 2.3167x; 1.0028x over previous
import functools

import jax
import jax.numpy as jnp
from jax import lax
from jax.experimental import pallas as pl
from jax.experimental.pallas import tpu as pltpu
from jax.experimental.pallas import tpu_sc as plsc

B = 16384
D = 32
H = 128
NC = 2
NS = 16
NW = NC * NS
BPW = B // NW

NU = 1000000
CW = 768
NFULL = NU // CW
PART_R0 = NFULL * CW
PART_W = NU - PART_R0
W_HI = 22
CPW_HI = 41
CPW_LO = 40
PAIRS = (CPW_HI + 1) // 2
CAP = 96
TRASH = B
LCAP = B + 16
SENTINEL = 0x7FFFFFF0


def _iota16():
    return lax.iota(jnp.int32, 16)


def _full16(v):
    return jnp.full((16,), v, jnp.int32)


def _sc_user_body(u_idx, ut, out, uidx_v, jl, rl, chunk0, chunk1, partbuf,
                  staging, jslots, sem_c0, sem_c1, sem_s):
    wid = lax.axis_index("s") * NC + lax.axis_index("c")
    chunk_lo = jnp.where(wid < W_HI, wid * CPW_HI,
                         W_HI * CPW_HI + (wid - W_HI) * CPW_LO)
    n_full = jnp.where(wid < W_HI, CPW_HI, CPW_LO)
    r_lo = chunk_lo * CW
    r_hi = r_lo + n_full * CW + jnp.where(wid == NW - 1, PART_W, 0)
    bufs = (chunk0, chunk1)

    sems = (sem_c0, sem_c1)
    pltpu.async_copy(ut.at[:, pl.ds(chunk_lo * CW, CW)], chunk0, sem_c0)
    pltpu.async_copy(ut.at[:, pl.ds((chunk_lo + 1) * CW, CW)], chunk1, sem_c1)

    pltpu.sync_copy(u_idx, uidx_v)

    def sel_body(c, cnt):
        v = uidx_v[pl.ds(c * 16, 16)]
        j = _iota16() + c * 16
        m = (v >= r_lo) & (v < r_hi)
        plsc.store_compressed(jl.at[pl.ds(cnt, 16)], j, mask=m)
        plsc.store_compressed(rl.at[pl.ds(cnt, 16)], v, mask=m)
        return cnt + plsc.all_reduce_population_count(m)[0]

    cnt_list = lax.fori_loop(0, B // 16, sel_body, jnp.int32(0))
    rl[pl.ds(cnt_list, 16)] = _full16(SENTINEL)

    for q in range(CAP // 16):
        jslots[pl.ds(q * 16, 16)] = _full16(TRASH)

    def flush():
        pltpu.async_copy(staging, out.at[jslots], sem_s).wait()
        for q in range(CAP // 16):
            jslots[pl.ds(q * 16, 16)] = _full16(TRASH)

    ngroups = (cnt_list + 15) // 16

    def extract(buf, width, r0, cs):
        def gbody(g, cs):
            rv = rl[pl.ds(g * 16, 16)]
            m = (rv >= r0) & (rv < r0 + width)
            npc = plsc.all_reduce_population_count(m)[0]

            @pl.when(npc > 0)
            def _():
                jv = jl[pl.ds(g * 16, 16)]
                pos = plsc.cumsum(jnp.where(m, 1, 0).astype(jnp.int32))
                slots = cs + pos - 1
                col = jnp.where(m, rv - r0, 0)
                for d in range(D):
                    vals = plsc.load_gather(buf, [_full16(d), col], mask=m)
                    plsc.store_scatter(staging, [slots, _full16(d)], vals,
                                       mask=m)
                plsc.store_scatter(jslots, [slots], jv, mask=m)

            cs = cs + npc

            def do_flush(c):
                flush()
                return jnp.int32(0)

            return lax.cond(cs >= CAP - 16, do_flush, lambda c: c, cs)

        return lax.fori_loop(0, ngroups, gbody, cs)

    def pair_body(p, cs):
        for b in range(2):
            c = 2 * p + b
            buf = bufs[b]
            sem_b = sems[b]

            def process(cs):
                pltpu.make_async_copy(
                    ut.at[:, pl.ds(0, CW)], buf, sem_b).wait()
                cs = extract(buf, CW, (chunk_lo + c) * CW, cs)

                @pl.when(c + 2 < n_full)
                def _():
                    pltpu.async_copy(
                        ut.at[:, pl.ds((chunk_lo + c + 2) * CW, CW)],
                        buf, sem_b)

                return cs

            cs = lax.cond(c < n_full, process, lambda c_: c_, cs)
        return cs

    cs = lax.fori_loop(0, PAIRS, pair_body, jnp.int32(0))

    def do_partial(cs):
        pltpu.sync_copy(ut.at[:, pl.ds(PART_R0, PART_W)], partbuf)
        return extract(partbuf, PART_W, jnp.int32(PART_R0), cs)

    cs = lax.cond(wid == NW - 1, do_partial, lambda c: c, cs)
    flush()


def _sc_user(u_idx, ut):
    mesh = plsc.VectorSubcoreMesh(core_axis_name="c", subcore_axis_name="s")
    scratch = [
        pltpu.VMEM((B,), jnp.int32),
        pltpu.VMEM((LCAP,), jnp.int32),
        pltpu.VMEM((LCAP,), jnp.int32),
        pltpu.VMEM((D, CW), jnp.float32),
        pltpu.VMEM((D, CW), jnp.float32),
        pltpu.VMEM((D, PART_W), jnp.float32),
        pltpu.VMEM((CAP, H), jnp.float32),
        pltpu.VMEM((CAP,), jnp.int32),
        pltpu.SemaphoreType.DMA,
        pltpu.SemaphoreType.DMA,
        pltpu.SemaphoreType.DMA,
    ]
    k = pl.kernel(
        _sc_user_body,
        out_type=jax.ShapeDtypeStruct((B + 2048, H), jnp.float32),
        mesh=mesh,
        scratch_types=scratch,
        compiler_params=pltpu.CompilerParams(needs_layout_passes=False),
    )
    return k(u_idx, ut)


def _sc_small_body(i_idx, l_idx, c_idx, it, lt, ct, out, idxb, rows, sem):
    wid = lax.axis_index("s") * NC + lax.axis_index("c")
    base = wid * BPW
    idxs = (i_idx, l_idx, c_idx)
    tables = (it, lt, ct)
    for t in range(3):
        pltpu.sync_copy(idxs[t].at[pl.ds(base, BPW)], idxb)
        pltpu.async_copy(tables[t].at[idxb], rows, sem).wait()
        pltpu.sync_copy(rows, out.at[t, pl.ds(base, BPW)])


def _sc_small(i_idx, l_idx, c_idx, it, lt, ct):
    mesh = plsc.VectorSubcoreMesh(core_axis_name="c", subcore_axis_name="s")
    scratch = [
        pltpu.VMEM((BPW,), jnp.int32),
        pltpu.VMEM((BPW, D), jnp.float32),
        pltpu.SemaphoreType.DMA,
    ]
    k = pl.kernel(
        _sc_small_body,
        out_type=jax.ShapeDtypeStruct((3, B, D), jnp.float32),
        mesh=mesh,
        scratch_types=scratch,
        compiler_params=pltpu.CompilerParams(use_tc_tiling_on_sc=False),
    )
    return k(i_idx, l_idx, c_idx, it, lt, ct)


BM = 2048


def _mlp_body(gu_ref, g3_ref, w1_ref, b1_ref, w2_ref, b2_ref, out_ref):
    h = jnp.dot(gu_ref[:, 0:D], w1_ref[0:D, :],
                preferred_element_type=jnp.float32)
    for t in range(3):
        h = h + jnp.dot(g3_ref[t], w1_ref[(t + 1) * D:(t + 2) * D, :],
                        preferred_element_type=jnp.float32)
    h = jnp.maximum(h + b1_ref[0, :][None, :], 0.0)
    out_ref[...] = (
        jnp.sum(h * w2_ref[0, :][None, :], axis=1, keepdims=True)
        + b2_ref[0, 0]
    )


def _mlp(gu, g3, W1, b1, W2, b2):
    w2_row = W2.reshape(1, H)
    b1_row = b1.reshape(1, H)
    b2_s = b2.reshape(1, 1)
    out = pl.pallas_call(
        _mlp_body,
        grid=(B // BM,),
        in_specs=[
            pl.BlockSpec((BM, H), lambda i: (i, 0)),
            pl.BlockSpec((3, BM, D), lambda i: (0, i, 0)),
            pl.BlockSpec((H, H), lambda i: (0, 0)),
            pl.BlockSpec((1, H), lambda i: (0, 0)),
            pl.BlockSpec((1, H), lambda i: (0, 0)),
            pl.BlockSpec((1, 1), lambda i: (0, 0)),
        ],
        out_specs=pl.BlockSpec((BM, 1), lambda i: (i, 0)),
        out_shape=jax.ShapeDtypeStruct((B, 1), jnp.float32),
    )(gu, g3, W1, b1_row, w2_row, b2_s)
    return out[:, 0]


def kernel(user, item, language, category,
           user_emb, item_emb, language_emb, category_emb,
           W1, b1, W2, b2):
    user = user.astype(jnp.int32)
    item = item.astype(jnp.int32)
    language = language.astype(jnp.int32)
    category = category.astype(jnp.int32)
    gu = _sc_user(user, user_emb.T)
    g3 = _sc_small(item, language, category,
                   item_emb, language_emb, category_emb)
    return _mlp(gu, g3, W1, b1, W2, b2)
